# speedup vs baseline: 1.0733x; 1.0733x over previous
import functools

import jax
import jax.numpy as jnp
from jax.experimental import pallas as pl
from jax.experimental.pallas import tpu as pltpu

N = 10000
E = 320000
IN_DIM = 128
H = 64
L = 8
EPS = 0.1
OUT = 64

BLK = 1000
GRID = N // BLK


def _dotT(x, w):
    return jax.lax.dot_general(x, w, (((1,), (1,)), ((), ())),
                               preferred_element_type=jnp.float32)


def _dot(x, w):
    return jax.lax.dot_general(x, w, (((1,), (0,)), ((), ())),
                               preferred_element_type=jnp.float32)


def _embed_body(x_ref, w_ref, b_ref, o_ref):
    o_ref[...] = _dotT(x_ref[...], w_ref[...]) + b_ref[...]


def _a_body(xp_ref, agg_ref, w_ref, v_ref, b_ref, o_ref):
    o_ref[...] = jnp.tanh(_dotT(xp_ref[...], w_ref[...]) +
                          _dotT(agg_ref[...], v_ref[...]) + b_ref[...])


def _upd_body(other_ref, a_ref, back_ref, w_ref, v_ref, o_ref, *, coef):
    o_ref[...] = other_ref[...] + coef * (_dot(a_ref[...], w_ref[...]) +
                                          _dot(back_ref[...], v_ref[...]))


def _readout_body(p_ref, q_ref, w1a_ref, w1b_ref, b1_ref, w2_ref, b2_ref, o_ref):
    h1 = (_dotT(p_ref[...], w1a_ref[...]) + _dotT(q_ref[...], w1b_ref[...])
          + b1_ref[...])
    h1 = jnp.where(h1 >= 0, h1, 0.01 * h1)
    h2 = _dotT(h1, w2_ref[...]) + b2_ref[...]
    o_ref[...] = jnp.where(h2 >= 0, h2, 0.01 * h2)


def _row_spec(cols):
    return pl.BlockSpec((BLK, cols), lambda i: (i, 0))


def _full_spec(shape):
    return pl.BlockSpec(shape, lambda i: tuple(0 for _ in shape))


def _tc_call(body, out_cols, row_args, full_args):
    in_specs = ([_row_spec(a.shape[1]) for a in row_args] +
                [_full_spec(a.shape) for a in full_args])
    return pl.pallas_call(
        body,
        grid=(GRID,),
        in_specs=in_specs,
        out_specs=_row_spec(out_cols),
        out_shape=jax.ShapeDtypeStruct((N, out_cols), jnp.float32),
    )(*row_args, *full_args)


def _seg_sum(vals, idx):
    return jax.ops.segment_sum(vals, idx, num_segments=N)


def _grad_h(xpart, W, V, b2d, src, dst):
    agg = _seg_sum(xpart[src], dst)
    a = _tc_call(_a_body, H, (xpart, agg), (W, V, b2d))
    back = _seg_sum(a[dst], src)
    return a, back


def kernel(x, edge_index, batch, W_emb, b_emb, Wp, Vp, bp, Wq, Vq, bq,
           W1, b1, W2, b2):
    src = edge_index[0]
    dst = edge_index[1]
    b_emb2 = b_emb.reshape(1, H)
    bp2 = bp.reshape(1, H)
    bq2 = bq.reshape(1, H)
    b12 = b1.reshape(1, H)
    b22 = b2.reshape(1, OUT)

    h = _tc_call(_embed_body, H, (x,), (W_emb, b_emb2))
    p = h
    q = h

    upd_m = functools.partial(_upd_body, coef=-EPS)
    upd_p = functools.partial(_upd_body, coef=EPS)
    for _ in range(L):
        a, back = _grad_h(q, Wq, Vq, bq2, src, dst)
        p = _tc_call(upd_m, H, (p, a, back), (Wq, Vq))
        a, back = _grad_h(p, Wp, Vp, bp2, src, dst)
        q = _tc_call(upd_p, H, (q, a, back), (Wp, Vp))

    W1a = W1[:, :H]
    W1b = W1[:, H:]
    return _tc_call(_readout_body, OUT, (p, q), (W1a, W1b, b12, W2, b22))


# speedup vs baseline: 1.4749x; 1.3741x over previous
import functools

import jax
import jax.numpy as jnp
from jax import lax
from jax.experimental import pallas as pl
from jax.experimental.pallas import tpu as pltpu
from jax.experimental.pallas import tpu_sc as plsc

N = 10000
E = 320000
IN_DIM = 128
H = 64
L = 8
EPS = 0.1
OUT = 64

BLK = 1000
GRID = N // BLK

NC = 2
NS = 16
NW = NC * NS
EPT = E // NW
CH = 40
NCHUNK = EPT // CH


_sc_mesh = plsc.VectorSubcoreMesh(core_axis_name="c", subcore_axis_name="s")


@functools.partial(
    pl.kernel,
    out_type=jax.ShapeDtypeStruct((E, H), jnp.float32),
    mesh=_sc_mesh,
    compiler_params=pltpu.CompilerParams(use_tc_tiling_on_sc=False),
    scratch_types=[
        pltpu.VMEM((NCHUNK, CH), jnp.int32),
        pltpu.VMEM((2, CH, H), jnp.float32),
        pltpu.SemaphoreType.DMA,
        pltpu.SemaphoreType.DMA,
    ],
)
def _sc_gather(gidx_hbm, table_hbm, out_hbm, gi, rows, sem0, sem1):
    cid = lax.axis_index("c")
    sid = lax.axis_index("s")
    wid = cid * NS + sid
    ebase = wid * EPT

    pltpu.sync_copy(gidx_hbm.at[wid], gi)

    sems = (sem0, sem1)

    def start(c, b):
        pltpu.async_copy(table_hbm.at[gi.at[c]], rows.at[b], sems[b])

    def wait(b):
        pltpu.make_async_copy(table_hbm.at[pl.ds(0, CH)], rows.at[b],
                              sems[b]).wait()

    def flush(c, b):
        pltpu.sync_copy(rows.at[b], out_hbm.at[pl.ds(ebase + c * CH, CH)])

    start(0, 0)
    start(1, 1)

    def pair(i, carry):
        c0 = 2 * i
        wait(0)
        flush(c0, 0)

        @pl.when(c0 + 2 < NCHUNK)
        def _():
            start(c0 + 2, 0)

        wait(1)
        flush(c0 + 1, 1)

        @pl.when(c0 + 3 < NCHUNK)
        def _():
            start(c0 + 3, 1)

        return carry

    lax.fori_loop(0, NCHUNK // 2, pair, 0)


def _dotT(x, w):
    return jax.lax.dot_general(x, w, (((1,), (1,)), ((), ())),
                               preferred_element_type=jnp.float32)


def _dot(x, w):
    return jax.lax.dot_general(x, w, (((1,), (0,)), ((), ())),
                               preferred_element_type=jnp.float32)


def _embed_body(x_ref, w_ref, b_ref, o_ref):
    o_ref[...] = _dotT(x_ref[...], w_ref[...]) + b_ref[...]


def _a_body(xp_ref, agg_ref, w_ref, v_ref, b_ref, o_ref):
    o_ref[...] = jnp.tanh(_dotT(xp_ref[...], w_ref[...]) +
                          _dotT(agg_ref[...], v_ref[...]) + b_ref[...])


def _upd_body(other_ref, a_ref, back_ref, w_ref, v_ref, o_ref, *, coef):
    o_ref[...] = other_ref[...] + coef * (_dot(a_ref[...], w_ref[...]) +
                                          _dot(back_ref[...], v_ref[...]))


def _readout_body(p_ref, q_ref, w1a_ref, w1b_ref, b1_ref, w2_ref, b2_ref, o_ref):
    h1 = (_dotT(p_ref[...], w1a_ref[...]) + _dotT(q_ref[...], w1b_ref[...])
          + b1_ref[...])
    h1 = jnp.where(h1 >= 0, h1, 0.01 * h1)
    h2 = _dotT(h1, w2_ref[...]) + b2_ref[...]
    o_ref[...] = jnp.where(h2 >= 0, h2, 0.01 * h2)


def _spec(a):
    if a.shape[0] == N:
        return pl.BlockSpec((BLK, a.shape[1]), lambda i: (i, 0))
    return pl.BlockSpec(a.shape, lambda i: tuple(0 for _ in a.shape))


def _tc_call(body, out_cols, *args):
    return pl.pallas_call(
        body,
        grid=(GRID,),
        in_specs=[_spec(a) for a in args],
        out_specs=pl.BlockSpec((BLK, out_cols), lambda i: (i, 0)),
        out_shape=jax.ShapeDtypeStruct((N, out_cols), jnp.float32),
    )(*args)


def kernel(x, edge_index, batch, W_emb, b_emb, Wp, Vp, bp, Wq, Vq, bq,
           W1, b1, W2, b2):
    src3 = edge_index[0].reshape(NW, NCHUNK, CH)
    dst3 = edge_index[1].reshape(NW, NCHUNK, CH)
    src = edge_index[0]
    dst = edge_index[1]
    b_emb2 = b_emb.reshape(1, H)
    bp2 = bp.reshape(1, H)
    bq2 = bq.reshape(1, H)
    b12 = b1.reshape(1, H)
    b22 = b2.reshape(1, OUT)

    h = _tc_call(_embed_body, H, x, W_emb, b_emb2)
    p = h
    q = h

    upd_m = functools.partial(_upd_body, coef=-EPS)
    upd_p = functools.partial(_upd_body, coef=EPS)

    def grad_h(xpart, W, V, b2d):
        agg = jax.ops.segment_sum(_sc_gather(src3, xpart), dst,
                                  num_segments=N)
        a = _tc_call(_a_body, H, xpart, agg, W, V, b2d)
        back = jax.ops.segment_sum(_sc_gather(dst3, a), src,
                                   num_segments=N)
        return a, back

    for _ in range(L):
        a, back = grad_h(q, Wq, Vq, bq2)
        p = _tc_call(upd_m, H, p, a, back, Wq, Vq)
        a, back = grad_h(p, Wp, Vp, bp2)
        q = _tc_call(upd_p, H, q, a, back, Wp, Vp)

    W1a = W1[:, :H]
    W1b = W1[:, H:]
    return _tc_call(_readout_body, OUT, p, q, W1a, W1b, b12, W2, b22)


# speedup vs baseline: 1.5836x; 1.0737x over previous
import functools

import jax
import jax.numpy as jnp
from jax import lax
from jax.experimental import pallas as pl
from jax.experimental.pallas import tpu as pltpu
from jax.experimental.pallas import tpu_sc as plsc

N = 10000
E = 320000
IN_DIM = 128
H = 64
L = 8
EPS = 0.1
OUT = 64

BLK = 1000
GRID = N // BLK

NC = 2
NS = 16
NW = NC * NS
EPT = E // NW
CH = 40
NCHUNK = EPT // CH


_sc_mesh = plsc.VectorSubcoreMesh(core_axis_name="c", subcore_axis_name="s")


NBUF = 10
LOOKAHEAD = 5


@functools.partial(
    pl.kernel,
    out_type=jax.ShapeDtypeStruct((E, H), jnp.float32),
    mesh=_sc_mesh,
    compiler_params=pltpu.CompilerParams(use_tc_tiling_on_sc=False),
    scratch_types=(
        [pltpu.VMEM((NCHUNK, CH), jnp.int32),
         pltpu.VMEM((NBUF, CH, H), jnp.float32)]
        + [pltpu.SemaphoreType.DMA] * (2 * NBUF)
    ),
)
def _sc_gather(gidx_hbm, table_hbm, out_hbm, gi, rows, *sems):
    gsem = sems[:NBUF]
    fsem = sems[NBUF:]
    cid = lax.axis_index("c")
    sid = lax.axis_index("s")
    wid = cid * NS + sid
    ebase = wid * EPT

    pltpu.sync_copy(gidx_hbm.at[wid], gi)

    def start_gather(c, b):
        pltpu.async_copy(table_hbm.at[gi.at[c]], rows.at[b], gsem[b])

    def wait_gather(b):
        pltpu.make_async_copy(table_hbm.at[pl.ds(0, CH)], rows.at[b],
                              gsem[b]).wait()

    def start_flush(c, b):
        pltpu.async_copy(rows.at[b], out_hbm.at[pl.ds(ebase + c * CH, CH)],
                         fsem[b])

    def wait_flush(b):
        pltpu.make_async_copy(rows.at[b], out_hbm.at[pl.ds(ebase, CH)],
                              fsem[b]).wait()

    for b in range(LOOKAHEAD):
        start_gather(b, b)

    def outer(i, carry):
        for j in range(NBUF):
            v = NBUF * i + j
            bg = (j + LOOKAHEAD) % NBUF

            @pl.when(v + LOOKAHEAD < NCHUNK)
            def _():
                @pl.when(v + LOOKAHEAD >= NBUF)
                def _():
                    wait_flush(bg)

                start_gather(v + LOOKAHEAD, bg)

            wait_gather(j)
            start_flush(v, j)
        return carry

    lax.fori_loop(0, NCHUNK // NBUF, outer, 0)

    for b in range(NBUF):
        wait_flush(b)


def _dotT(x, w):
    return jax.lax.dot_general(x, w, (((1,), (1,)), ((), ())),
                               preferred_element_type=jnp.float32)


def _dot(x, w):
    return jax.lax.dot_general(x, w, (((1,), (0,)), ((), ())),
                               preferred_element_type=jnp.float32)


def _embed_body(x_ref, w_ref, b_ref, o_ref):
    o_ref[...] = _dotT(x_ref[...], w_ref[...]) + b_ref[...]


def _a_body(xp_ref, agg_ref, w_ref, v_ref, b_ref, o_ref):
    o_ref[...] = jnp.tanh(_dotT(xp_ref[...], w_ref[...]) +
                          _dotT(agg_ref[...], v_ref[...]) + b_ref[...])


def _upd_body(other_ref, a_ref, back_ref, w_ref, v_ref, o_ref, *, coef):
    o_ref[...] = other_ref[...] + coef * (_dot(a_ref[...], w_ref[...]) +
                                          _dot(back_ref[...], v_ref[...]))


def _readout_body(p_ref, q_ref, w1a_ref, w1b_ref, b1_ref, w2_ref, b2_ref, o_ref):
    h1 = (_dotT(p_ref[...], w1a_ref[...]) + _dotT(q_ref[...], w1b_ref[...])
          + b1_ref[...])
    h1 = jnp.where(h1 >= 0, h1, 0.01 * h1)
    h2 = _dotT(h1, w2_ref[...]) + b2_ref[...]
    o_ref[...] = jnp.where(h2 >= 0, h2, 0.01 * h2)


def _spec(a):
    if a.shape[0] == N:
        return pl.BlockSpec((BLK, a.shape[1]), lambda i: (i, 0))
    return pl.BlockSpec(a.shape, lambda i: tuple(0 for _ in a.shape))


def _tc_call(body, out_cols, *args):
    return pl.pallas_call(
        body,
        grid=(GRID,),
        in_specs=[_spec(a) for a in args],
        out_specs=pl.BlockSpec((BLK, out_cols), lambda i: (i, 0)),
        out_shape=jax.ShapeDtypeStruct((N, out_cols), jnp.float32),
    )(*args)


def kernel(x, edge_index, batch, W_emb, b_emb, Wp, Vp, bp, Wq, Vq, bq,
           W1, b1, W2, b2):
    src3 = edge_index[0].reshape(NW, NCHUNK, CH)
    dst3 = edge_index[1].reshape(NW, NCHUNK, CH)
    src = edge_index[0]
    dst = edge_index[1]
    b_emb2 = b_emb.reshape(1, H)
    bp2 = bp.reshape(1, H)
    bq2 = bq.reshape(1, H)
    b12 = b1.reshape(1, H)
    b22 = b2.reshape(1, OUT)

    h = _tc_call(_embed_body, H, x, W_emb, b_emb2)
    p = h
    q = h

    upd_m = functools.partial(_upd_body, coef=-EPS)
    upd_p = functools.partial(_upd_body, coef=EPS)

    def grad_h(xpart, W, V, b2d):
        agg = jax.ops.segment_sum(_sc_gather(src3, xpart), dst,
                                  num_segments=N)
        a = _tc_call(_a_body, H, xpart, agg, W, V, b2d)
        back = jax.ops.segment_sum(_sc_gather(dst3, a), src,
                                   num_segments=N)
        return a, back

    for _ in range(L):
        a, back = grad_h(q, Wq, Vq, bq2)
        p = _tc_call(upd_m, H, p, a, back, Wq, Vq)
        a, back = grad_h(p, Wp, Vp, bp2)
        q = _tc_call(upd_p, H, q, a, back, Wp, Vp)

    W1a = W1[:, :H]
    W1b = W1[:, H:]
    return _tc_call(_readout_body, OUT, p, q, W1a, W1b, b12, W2, b22)


# speedup vs baseline: 1.5861x; 1.0016x over previous
import functools

import jax
import jax.numpy as jnp
from jax import lax
from jax.experimental import pallas as pl
from jax.experimental.pallas import tpu as pltpu
from jax.experimental.pallas import tpu_sc as plsc

N = 10000
E = 320000
IN_DIM = 128
H = 64
L = 8
EPS = 0.1
OUT = 64

BLK = 1000
GRID = N // BLK

NC = 2
NS = 16
NW = NC * NS
EPT = E // NW
CH = 40
NCHUNK = EPT // CH


_sc_mesh = plsc.VectorSubcoreMesh(core_axis_name="c", subcore_axis_name="s")


NBUF = 10
LOOKAHEAD = 8


@functools.partial(
    pl.kernel,
    out_type=jax.ShapeDtypeStruct((E, H), jnp.float32),
    mesh=_sc_mesh,
    compiler_params=pltpu.CompilerParams(use_tc_tiling_on_sc=False),
    scratch_types=(
        [pltpu.VMEM((NCHUNK, CH), jnp.int32),
         pltpu.VMEM((NBUF, CH, H), jnp.float32)]
        + [pltpu.SemaphoreType.DMA] * (2 * NBUF)
    ),
)
def _sc_gather(gidx_hbm, table_hbm, out_hbm, gi, rows, *sems):
    gsem = sems[:NBUF]
    fsem = sems[NBUF:]
    cid = lax.axis_index("c")
    sid = lax.axis_index("s")
    wid = cid * NS + sid
    ebase = wid * EPT

    pltpu.sync_copy(gidx_hbm.at[wid], gi)

    def start_gather(c, b):
        pltpu.async_copy(table_hbm.at[gi.at[c]], rows.at[b], gsem[b])

    def wait_gather(b):
        pltpu.make_async_copy(table_hbm.at[pl.ds(0, CH)], rows.at[b],
                              gsem[b]).wait()

    def start_flush(c, b):
        pltpu.async_copy(rows.at[b], out_hbm.at[pl.ds(ebase + c * CH, CH)],
                         fsem[b])

    def wait_flush(b):
        pltpu.make_async_copy(rows.at[b], out_hbm.at[pl.ds(ebase, CH)],
                              fsem[b]).wait()

    for b in range(LOOKAHEAD):
        start_gather(b, b)

    def outer(i, carry):
        for j in range(NBUF):
            v = NBUF * i + j
            bg = (j + LOOKAHEAD) % NBUF

            @pl.when(v + LOOKAHEAD < NCHUNK)
            def _():
                @pl.when(v + LOOKAHEAD >= NBUF)
                def _():
                    wait_flush(bg)

                start_gather(v + LOOKAHEAD, bg)

            wait_gather(j)
            start_flush(v, j)
        return carry

    lax.fori_loop(0, NCHUNK // NBUF, outer, 0)

    for b in range(NBUF):
        wait_flush(b)


def _dotT(x, w):
    return jax.lax.dot_general(x, w, (((1,), (1,)), ((), ())),
                               preferred_element_type=jnp.float32)


def _dot(x, w):
    return jax.lax.dot_general(x, w, (((1,), (0,)), ((), ())),
                               preferred_element_type=jnp.float32)


def _embed_body(x_ref, w_ref, b_ref, o_ref):
    o_ref[...] = _dotT(x_ref[...], w_ref[...]) + b_ref[...]


def _a_body(xp_ref, agg_ref, w_ref, v_ref, b_ref, o_ref):
    o_ref[...] = jnp.tanh(_dotT(xp_ref[...], w_ref[...]) +
                          _dotT(agg_ref[...], v_ref[...]) + b_ref[...])


def _upd_body(other_ref, a_ref, back_ref, w_ref, v_ref, o_ref, *, coef):
    o_ref[...] = other_ref[...] + coef * (_dot(a_ref[...], w_ref[...]) +
                                          _dot(back_ref[...], v_ref[...]))


def _readout_body(p_ref, q_ref, w1a_ref, w1b_ref, b1_ref, w2_ref, b2_ref, o_ref):
    h1 = (_dotT(p_ref[...], w1a_ref[...]) + _dotT(q_ref[...], w1b_ref[...])
          + b1_ref[...])
    h1 = jnp.where(h1 >= 0, h1, 0.01 * h1)
    h2 = _dotT(h1, w2_ref[...]) + b2_ref[...]
    o_ref[...] = jnp.where(h2 >= 0, h2, 0.01 * h2)


def _spec(a):
    if a.shape[0] == N:
        return pl.BlockSpec((BLK, a.shape[1]), lambda i: (i, 0))
    return pl.BlockSpec(a.shape, lambda i: tuple(0 for _ in a.shape))


def _tc_call(body, out_cols, *args):
    return pl.pallas_call(
        body,
        grid=(GRID,),
        in_specs=[_spec(a) for a in args],
        out_specs=pl.BlockSpec((BLK, out_cols), lambda i: (i, 0)),
        out_shape=jax.ShapeDtypeStruct((N, out_cols), jnp.float32),
    )(*args)


def kernel(x, edge_index, batch, W_emb, b_emb, Wp, Vp, bp, Wq, Vq, bq,
           W1, b1, W2, b2):
    src3 = edge_index[0].reshape(NW, NCHUNK, CH)
    dst3 = edge_index[1].reshape(NW, NCHUNK, CH)
    src = edge_index[0]
    dst = edge_index[1]
    b_emb2 = b_emb.reshape(1, H)
    bp2 = bp.reshape(1, H)
    bq2 = bq.reshape(1, H)
    b12 = b1.reshape(1, H)
    b22 = b2.reshape(1, OUT)

    h = _tc_call(_embed_body, H, x, W_emb, b_emb2)
    p = h
    q = h

    upd_m = functools.partial(_upd_body, coef=-EPS)
    upd_p = functools.partial(_upd_body, coef=EPS)

    def grad_h(xpart, W, V, b2d):
        agg = jax.ops.segment_sum(_sc_gather(src3, xpart), dst,
                                  num_segments=N)
        a = _tc_call(_a_body, H, xpart, agg, W, V, b2d)
        back = jax.ops.segment_sum(_sc_gather(dst3, a), src,
                                   num_segments=N)
        return a, back

    for _ in range(L):
        a, back = grad_h(q, Wq, Vq, bq2)
        p = _tc_call(upd_m, H, p, a, back, Wq, Vq)
        a, back = grad_h(p, Wp, Vp, bp2)
        q = _tc_call(upd_p, H, q, a, back, Wp, Vp)

    W1a = W1[:, :H]
    W1b = W1[:, H:]
    return _tc_call(_readout_body, OUT, p, q, W1a, W1b, b12, W2, b22)
